# baseline (speedup 1.0000x reference)
import jax
import jax.numpy as jnp
from jax.experimental import pallas as pl
from jax.experimental.pallas import tpu as pltpu

_T = 8
_D = 10
_DP = 16
_H = 16
_HID = 32
_BBLK = 8192
_NCH = 128 // _T
_W = _BBLK * _T // 128
_G = 2
_CG = _NCH // _G


def _lstm_head_kernel(x_ref, wfb_ref, whh_ref, t1_ref, mrev_ref,
                      whf_ref, wo_ref, bo_ref, out_ref):
    H = _H
    T = _T

    xb = x_ref[...].reshape(_BBLK * T // 128, 128)
    idx3 = jnp.transpose(xb).astype(jnp.int32).reshape(_NCH, T, _W)

    dig = jax.lax.broadcasted_iota(jnp.int32, (_CG, _DP, _W), 1)

    def onehot(g, t):
        rows = idx3[g * _CG:(g + 1) * _CG, t:t + 1, :]
        oh = ((dig == rows) | (dig == _D)).astype(jnp.bfloat16)
        return oh.reshape(_CG * _DP, _W)

    def gslice(w, g, rh, rw):
        return w[g * rh:(g + 1) * rh, g * rw:(g + 1) * rw]

    hs, cs, ms = [], [], []
    for g in range(_G):
        t1res = jnp.dot(gslice(t1_ref[...], g, _CG * 2 * H, _CG * _DP),
                        onehot(g, 0),
                        preferred_element_type=jnp.float32)
        t3 = t1res.reshape(_CG, 2 * H, _W)
        hs.append(t3[:, 0:H, :].reshape(_CG * H, _W))
        cs.append(t3[:, H:2 * H, :])
        ms.append(jnp.dot(gslice(mrev_ref[...], g, _CG * _HID, _CG * _DP),
                          onehot(g, T - 1),
                          preferred_element_type=jnp.float32))

    for t in range(1, T):
        for g in range(_G):
            gk = jnp.dot(gslice(wfb_ref[...], g, _CG * 4 * H, _CG * _DP),
                         onehot(g, t),
                         preferred_element_type=jnp.float32) + jnp.dot(
                gslice(whh_ref[...], g, _CG * 4 * H, _CG * H),
                hs[g].astype(jnp.bfloat16),
                preferred_element_type=jnp.float32)
            g3 = gk.reshape(_CG, 4 * H, _W)
            s = 0.5 + 0.5 * jnp.tanh(g3[:, 0:3 * H, :])
            gc = jnp.tanh(g3[:, 3 * H:4 * H, :])
            cs[g] = s[:, H:2 * H, :] * cs[g] + s[:, 0:H, :] * gc
            h3 = s[:, 2 * H:3 * H, :] * jnp.tanh(cs[g])
            hs[g] = h3.reshape(_CG * H, _W)

    ys = []
    for g in range(_G):
        hid = jnp.maximum(
            jnp.dot(gslice(whf_ref[...], g, _CG * _HID, _CG * H),
                    hs[g].astype(jnp.bfloat16),
                    preferred_element_type=jnp.float32) + ms[g], 0.0)
        logits = jnp.dot(wo_ref[...][g * _CG:(g + 1) * _CG,
                                     g * _CG * _HID:(g + 1) * _CG * _HID],
                         hid.astype(jnp.bfloat16),
                         preferred_element_type=jnp.float32)
        ys.append(logits)
    logits_all = jnp.concatenate(ys, axis=0) + bo_ref[...]
    out_ref[...] = (0.5 + 0.5 * jnp.tanh(logits_all)).reshape(1, _NCH, _W)


def _reorder(w):
    H = _H
    return jnp.concatenate([w[0:2 * H], w[3 * H:4 * H], w[2 * H:3 * H]], axis=0)


def _bd(w):
    return jnp.kron(jnp.eye(_NCH, dtype=jnp.float32), w).astype(jnp.bfloat16)


def _pad_oh(w):
    return jnp.concatenate(
        [w, jnp.zeros((w.shape[0], _DP - w.shape[1]), jnp.float32)], axis=1)


@jax.jit
def _forward(x, w_ih_f, w_hh_f, b_ih_f, b_hh_f, w_ih_r, w_hh_r, b_ih_r,
             b_hh_r, w_hid, b_hid, w_out, b_out):
    H = _H
    T = _T

    wf = _reorder(w_ih_f)
    bf = _reorder((b_ih_f + b_hh_f).reshape(4 * H, 1))
    half = jnp.concatenate([jnp.full((3 * H, 1), 0.5, jnp.float32),
                            jnp.ones((H, 1), jnp.float32)], axis=0)
    wfb = _bd(_pad_oh(jnp.concatenate([wf, bf], axis=1) * half))
    whh = _bd(_reorder(w_hh_f) * half)

    g0 = wf + bf
    s0 = jax.nn.sigmoid(g0[0:3 * H])
    c1 = s0[0:H] * jnp.tanh(g0[3 * H:4 * H])
    h1 = s0[2 * H:3 * H] * jnp.tanh(c1)
    t1 = _bd(_pad_oh(jnp.concatenate([h1, c1], axis=0)))

    wr = _reorder(w_ih_r)
    br = _reorder((b_ih_r + b_hh_r).reshape(4 * H, 1))
    gr = wr + br
    sr = jax.nn.sigmoid(gr[0:3 * H])
    cr = sr[0:H] * jnp.tanh(gr[3 * H:4 * H])
    hr = sr[2 * H:3 * H] * jnp.tanh(cr)
    mrev = _bd(_pad_oh(jnp.concatenate(
        [jnp.dot(w_hid[:, H:2 * H], hr), b_hid.reshape(_HID, 1)], axis=1)))

    whf = _bd(w_hid[:, 0:H])
    wo = _bd(0.5 * w_out)
    bo = jnp.full((_NCH, 1), 0.5 * b_out[0], jnp.float32)

    x_idx = x.reshape(-1, T)
    B = x_idx.shape[0]
    b_pad = ((B + _BBLK - 1) // _BBLK) * _BBLK
    nblk = b_pad // _BBLK
    if b_pad != B:
        x_idx = jnp.zeros((b_pad, T), jnp.float32).at[:B].set(x_idx)
    x_in = x_idx.reshape(nblk, _BBLK * T // 128, 128)

    out = pl.pallas_call(
        _lstm_head_kernel,
        out_shape=jax.ShapeDtypeStruct((nblk, _NCH, _W), jnp.float32),
        grid=(nblk,),
        in_specs=[
            pl.BlockSpec((1, _BBLK * T // 128, 128), lambda i: (i, 0, 0)),
            pl.BlockSpec(wfb.shape, lambda i: (0, 0)),
            pl.BlockSpec(whh.shape, lambda i: (0, 0)),
            pl.BlockSpec(t1.shape, lambda i: (0, 0)),
            pl.BlockSpec(mrev.shape, lambda i: (0, 0)),
            pl.BlockSpec(whf.shape, lambda i: (0, 0)),
            pl.BlockSpec(wo.shape, lambda i: (0, 0)),
            pl.BlockSpec(bo.shape, lambda i: (0, 0)),
        ],
        out_specs=pl.BlockSpec((1, _NCH, _W), lambda i: (i, 0, 0)),
        compiler_params=pltpu.CompilerParams(
            dimension_semantics=("parallel",)),
    )(x_in, wfb, whh, t1, mrev, whf, wo, bo)

    out_flat = jnp.transpose(out, (0, 2, 1)).reshape(b_pad, 1)
    return out_flat[:B]


def kernel(x, w_ih_f, w_hh_f, b_ih_f, b_hh_f, w_ih_r, w_hh_r, b_ih_r, b_hh_r,
           w_hid, b_hid, w_out, b_out):
    return _forward(x, w_ih_f, w_hh_f, b_ih_f, b_hh_f, w_ih_r, w_hh_r,
                    b_ih_r, b_hh_r, w_hid, b_hid, w_out, b_out)


# speedup vs baseline: 1.6106x; 1.6106x over previous
import functools

import jax
import jax.numpy as jnp
from jax.experimental import pallas as pl
from jax.experimental.pallas import tpu as pltpu

_T = 8
_D = 10
_H = 16
_HID = 32
_BBLK = 8192


def _lstm_head_kernel(idx_ref, wfb_ref, whh_ref, t1_ref, mrev_ref,
                      whf_ref, wo_ref, bo_ref, out_ref, *, bblk):
    H = _H
    T = _T
    N = T * bblk

    idx = idx_ref[...].reshape(1, N)
    dig = jax.lax.broadcasted_iota(jnp.int32, (_D + 1, N), 0)
    oh = ((dig == idx) | (dig == _D)).astype(jnp.bfloat16)

    t1res = jnp.dot(t1_ref[...], oh[:, :bblk],
                    preferred_element_type=jnp.float32)
    mres = jnp.dot(mrev_ref[...], oh[:, (T - 1) * bblk:],
                   preferred_element_type=jnp.float32)

    whh = whh_ref[...]
    wfb = wfb_ref[...]
    half = bblk // 2
    hs = [t1res[0:H, 0:half], t1res[0:H, half:]]
    cs = [t1res[H:2 * H, 0:half], t1res[H:2 * H, half:]]
    for t in range(1, T):
        for k in (0, 1):
            lo = t * bblk + k * half
            g = jnp.dot(wfb, oh[:, lo:lo + half],
                        preferred_element_type=jnp.float32) + jnp.dot(
                whh, hs[k].astype(jnp.bfloat16),
                preferred_element_type=jnp.float32)
            s = 0.5 + 0.5 * jnp.tanh(g[0:3 * H])
            gc = jnp.tanh(g[3 * H:4 * H])
            cs[k] = s[H:2 * H] * cs[k] + s[0:H] * gc
            hs[k] = s[2 * H:3 * H] * jnp.tanh(cs[k])
    h = jnp.concatenate(hs, axis=1)

    hid = jnp.maximum(
        jnp.dot(whf_ref[...], h.astype(jnp.bfloat16),
                preferred_element_type=jnp.float32) + mres,
        0.0)
    logits = jnp.dot(wo_ref[...], hid.astype(jnp.bfloat16),
                     preferred_element_type=jnp.float32) + bo_ref[...]
    out_ref[...] = (0.5 + 0.5 * jnp.tanh(logits[0:1])).reshape(1, 1, bblk)


def _reorder(w):
    H = _H
    return jnp.concatenate([w[0:2 * H], w[3 * H:4 * H], w[2 * H:3 * H]], axis=0)


@jax.jit
def _forward(x, w_ih_f, w_hh_f, b_ih_f, b_hh_f, w_ih_r, w_hh_r, b_ih_r,
             b_hh_r, w_hid, b_hid, w_out, b_out):
    H = _H
    T = _T

    wf = _reorder(w_ih_f)
    bf = _reorder((b_ih_f + b_hh_f).reshape(4 * H, 1))
    half = jnp.concatenate([jnp.full((3 * H, 1), 0.5, jnp.float32),
                            jnp.ones((H, 1), jnp.float32)], axis=0)
    wfb = (jnp.concatenate([wf, bf], axis=1) * half).astype(jnp.bfloat16)
    whh = (_reorder(w_hh_f) * half).astype(jnp.bfloat16)

    g0 = wf + bf
    s0 = jax.nn.sigmoid(g0[0:3 * H])
    c1 = s0[0:H] * jnp.tanh(g0[3 * H:4 * H])
    h1 = s0[2 * H:3 * H] * jnp.tanh(c1)
    t1 = jnp.concatenate([h1, c1], axis=0)
    t1_aug = jnp.concatenate(
        [t1, jnp.zeros((2 * H, 1), jnp.float32)], axis=1).astype(jnp.bfloat16)

    wr = _reorder(w_ih_r)
    br = _reorder((b_ih_r + b_hh_r).reshape(4 * H, 1))
    gr = wr + br
    sr = jax.nn.sigmoid(gr[0:3 * H])
    cr = sr[0:H] * jnp.tanh(gr[3 * H:4 * H])
    hr = sr[2 * H:3 * H] * jnp.tanh(cr)
    mrev = jnp.dot(w_hid[:, H:2 * H], hr)
    mrev_aug = jnp.concatenate(
        [mrev, b_hid.reshape(_HID, 1)], axis=1).astype(jnp.bfloat16)

    whf = w_hid[:, 0:H].astype(jnp.bfloat16)
    wo8 = (0.5 * jnp.zeros((8, _HID), jnp.float32).at[0:1].set(w_out)
           ).astype(jnp.bfloat16)
    bo8 = 0.5 * jnp.zeros((8, 1), jnp.float32).at[0:1, 0].set(b_out)

    x_idx = x.reshape(-1, T).astype(jnp.int32)
    B = x_idx.shape[0]
    b_pad = ((B + _BBLK - 1) // _BBLK) * _BBLK
    nblk = b_pad // _BBLK
    if b_pad != B:
        x_idx = jnp.zeros((b_pad, T), jnp.int32).at[:B].set(x_idx)
    idx_in = jnp.transpose(x_idx.reshape(nblk, _BBLK, T), (0, 2, 1)).reshape(
        nblk, 1, T * _BBLK)

    body = functools.partial(_lstm_head_kernel, bblk=_BBLK)
    out = pl.pallas_call(
        body,
        out_shape=jax.ShapeDtypeStruct((nblk, 1, _BBLK), jnp.float32),
        grid=(nblk,),
        in_specs=[
            pl.BlockSpec((1, 1, T * _BBLK), lambda i: (i, 0, 0)),
            pl.BlockSpec((4 * H, _D + 1), lambda i: (0, 0)),
            pl.BlockSpec((4 * H, H), lambda i: (0, 0)),
            pl.BlockSpec((2 * H, _D + 1), lambda i: (0, 0)),
            pl.BlockSpec((_HID, _D + 1), lambda i: (0, 0)),
            pl.BlockSpec((_HID, H), lambda i: (0, 0)),
            pl.BlockSpec((8, _HID), lambda i: (0, 0)),
            pl.BlockSpec((8, 1), lambda i: (0, 0)),
        ],
        out_specs=pl.BlockSpec((1, 1, _BBLK), lambda i: (i, 0, 0)),
        compiler_params=pltpu.CompilerParams(
            dimension_semantics=("parallel",)),
    )(idx_in, wfb, whh, t1_aug, mrev_aug, whf, wo8, bo8)

    return out.reshape(b_pad, 1)[:B]


def kernel(x, w_ih_f, w_hh_f, b_ih_f, b_hh_f, w_ih_r, w_hh_r, b_ih_r, b_hh_r,
           w_hid, b_hid, w_out, b_out):
    return _forward(x, w_ih_f, w_hh_f, b_ih_f, b_hh_f, w_ih_r, w_hh_r,
                    b_ih_r, b_hh_r, w_hid, b_hid, w_out, b_out)
